# baseline (speedup 1.0000x reference)
import jax
import jax.numpy as jnp
from jax.experimental import pallas as pl
from jax.experimental.pallas import tpu as pltpu

B, T, D = 2, 2048, 2048
E = 8
O = 2048
I = D // E
C = 256
TB = 256
NT = T // TB
NS = B * NT
P0 = E


def _fused_kernel(w_ref, b_ref, x_ref, dm_ref, cb_ref, o_ref,
                  ws_scr, bs_scr, p_scr, s_scr):
    s = pl.program_id(0)

    @pl.when(s < P0)
    def _w_phase():
        wblk = w_ref[...]
        acc = wblk[:, 0:I]
        for j in range(1, E):
            acc = acc + wblk[:, j * I:(j + 1) * I]
        ws_scr[pl.ds(s, 1), :] = jnp.sum(acc, axis=0, keepdims=True)

        @pl.when(s == 0)
        def _():
            bs_scr[0, 0] = jnp.sum(b_ref[...])

    @pl.when((s >= P0) & (s < P0 + NS))
    def _x_phase():
        j = s - P0
        xb = x_ref[0]
        cols = []
        for e in range(E):
            we = ws_scr[e:e + 1, :]
            cols.append(jnp.sum(xb[:, e * I:(e + 1) * I] * we, axis=1,
                                keepdims=True))
        p_scr[pl.ds(j * TB, TB), :] = jnp.concatenate(cols, axis=1)

    @pl.when((s >= P0 + NS) & (s < P0 + 2 * NS))
    def _dm_phase():
        j = s - P0 - NS
        bb = j // NT
        dmb = dm_ref[0]
        pb = p_scr[pl.ds(j * TB, TB), :]
        parts = []
        for e in range(E):
            parts.append(jnp.sum(dmb[:, e * C:(e + 1) * C] * pb[:, e:e + 1],
                                 axis=0, keepdims=True))
        contrib = jnp.concatenate(parts, axis=1)
        init = (j % NT) == 0
        prev = jnp.where(init, bs_scr[0, 0], s_scr[pl.ds(bb, 1), :])
        s_scr[pl.ds(bb, 1), :] = prev + contrib

    @pl.when(s >= P0 + 2 * NS)
    def _cb_phase():
        j = s - P0 - 2 * NS
        bb = j // NT
        sb = s_scr[pl.ds(bb, 1), :]
        prod = cb_ref[0] * sb
        o_ref[...] = jnp.sum(prod, axis=1).reshape(1, 1, TB)


def kernel(x, combine_array, dispatch_mask, W, b):
    dm2 = dispatch_mask.reshape(B, T, E * C)
    cb2 = combine_array.reshape(B, T, E * C)
    b2 = b.reshape(E, O // E)

    def w_idx(s):
        return (jnp.minimum(s, P0 - 1), 0)

    def mk_idx(off):
        def idx(s):
            j = jnp.clip(s - off, 0, NS - 1)
            return (j // NT, j % NT, 0)
        return idx

    def out_idx(s):
        j = jnp.clip(s - P0 - 2 * NS, 0, NS - 1)
        return (j // NT, 0, j % NT)

    out = pl.pallas_call(
        _fused_kernel,
        grid=(P0 + 3 * NS,),
        in_specs=[
            pl.BlockSpec((O // E, D), w_idx),
            pl.BlockSpec((E, O // E), lambda s: (0, 0)),
            pl.BlockSpec((1, TB, D), mk_idx(P0)),
            pl.BlockSpec((1, TB, E * C), mk_idx(P0 + NS)),
            pl.BlockSpec((1, TB, E * C), mk_idx(P0 + 2 * NS)),
        ],
        out_specs=pl.BlockSpec((1, 1, TB), out_idx),
        out_shape=jax.ShapeDtypeStruct((B, 1, T), jnp.float32),
        scratch_shapes=[
            pltpu.VMEM((E, I), jnp.float32),
            pltpu.SMEM((1, 1), jnp.float32),
            pltpu.VMEM((B * T, E), jnp.float32),
            pltpu.VMEM((B, E * C), jnp.float32),
        ],
    )(W, b2, x, dm2, cb2)

    return out.reshape(B, T)


# speedup vs baseline: 1.7739x; 1.7739x over previous
import jax
import jax.numpy as jnp
from jax.experimental import pallas as pl
from jax.experimental.pallas import tpu as pltpu

B, T, D = 2, 2048, 2048
E = 8
O = 2048
I = D // E
C = 256
TB = 256
NT = T // TB
NS = B * NT
P0 = E


def _fused_kernel(w_ref, b_ref, x_ref, dm_ref, cb_ref, o_ref,
                  ws_scr, bs_scr, s_scr):
    s = pl.program_id(0)

    @pl.when(s < P0)
    def _w_phase():
        wblk = w_ref[...]
        acc = wblk[:, 0:I]
        for j in range(1, E):
            acc = acc + wblk[:, j * I:(j + 1) * I]
        ws_scr[pl.ds(s, 1), :] = jnp.sum(acc, axis=0, keepdims=True)

        @pl.when(s == 0)
        def _():
            bs_scr[0, 0] = jnp.sum(b_ref[...])

    @pl.when((s >= P0) & (s < P0 + NS))
    def _xdm_phase():
        j = s - P0
        bb = j // NT
        init = (j % NT) == 0
        xb = x_ref[0]
        dmb = dm_ref[0]
        for e in range(E):
            we = ws_scr[e:e + 1, :]
            p_e = jnp.sum(xb[:, e * I:(e + 1) * I] * we, axis=1,
                          keepdims=True)
            contrib = jnp.sum(dmb[:, e, :] * p_e, axis=0,
                              keepdims=True)
            row = bb * E + e
            prev = jnp.where(init, bs_scr[0, 0], s_scr[pl.ds(row, 1), :])
            s_scr[pl.ds(row, 1), :] = prev + contrib

    @pl.when(s >= P0 + NS)
    def _cb_phase():
        j = s - P0 - NS
        bb = j // NT
        cbb = cb_ref[0]
        acc = cbb[:, 0, :] * s_scr[pl.ds(bb * E, 1), :]
        for e in range(1, E):
            acc = acc + cbb[:, e, :] * s_scr[pl.ds(bb * E + e, 1), :]
        o_ref[...] = jnp.sum(acc, axis=1).reshape(1, 1, TB)


def kernel(x, combine_array, dispatch_mask, W, b):
    b2 = b.reshape(E, O // E)

    def w_idx(s):
        return (jnp.minimum(s, P0 - 1), 0)

    def mk_idx(off):
        def idx(s):
            j = jnp.clip(s - off, 0, NS - 1)
            return (j // NT, j % NT, 0, 0)
        return idx

    def x_idx(s):
        j = jnp.clip(s - P0, 0, NS - 1)
        return (j // NT, j % NT, 0)

    def out_idx(s):
        j = jnp.clip(s - P0 - NS, 0, NS - 1)
        return (j // NT, 0, j % NT)

    out = pl.pallas_call(
        _fused_kernel,
        grid=(P0 + 2 * NS,),
        in_specs=[
            pl.BlockSpec((O // E, D), w_idx),
            pl.BlockSpec((E, O // E), lambda s: (0, 0)),
            pl.BlockSpec((1, TB, D), x_idx),
            pl.BlockSpec((1, TB, E, C), mk_idx(P0)),
            pl.BlockSpec((1, TB, E, C), mk_idx(P0 + NS)),
        ],
        out_specs=pl.BlockSpec((1, 1, TB), out_idx),
        out_shape=jax.ShapeDtypeStruct((B, 1, T), jnp.float32),
        scratch_shapes=[
            pltpu.VMEM((E, I), jnp.float32),
            pltpu.SMEM((1, 1), jnp.float32),
            pltpu.VMEM((B * E, C), jnp.float32),
        ],
    )(W, b2, x, dispatch_mask, combine_array)

    return out.reshape(B, T)
